# baseline (device time: 137520 ns/iter reference)
import jax
import jax.numpy as jnp
from jax import lax
from jax.experimental import pallas as pl
from jax.experimental.pallas import tpu as pltpu

B, S, D = 1, 1024, 2048
H, Dh, Dr = 16, 128, 32
DC_SHARD = 128
SCALE = (Dh + Dr) ** -0.5

_VMEM = pl.BlockSpec(memory_space=pltpu.VMEM)


def _f32dot(a, b):
    return jnp.dot(a, b, preferred_element_type=jnp.float32)


def _kv_body(x_ref, wdkv_ref, wuk_ref, wuv_ref, k_ref, v_ref,
             c_mine, c_peer, wuk_peer, wuv_peer, send_sems, recv_sems):
    my_x = lax.axis_index("x")
    my_y = lax.axis_index("y")
    my_z = lax.axis_index("z")
    peer = (1 - my_x, my_y, my_z)

    barrier_sem = pltpu.get_barrier_semaphore()
    pl.semaphore_signal(barrier_sem, inc=1, device_id=peer,
                        device_id_type=pl.DeviceIdType.MESH)
    pl.semaphore_wait(barrier_sem, 1)

    c_mine[...] = _f32dot(x_ref[0], wdkv_ref[...])

    rdmas = []
    for i, (src, dst) in enumerate(
        [(c_mine, c_peer), (wuk_ref, wuk_peer), (wuv_ref, wuv_peer)]
    ):
        r = pltpu.make_async_remote_copy(
            src_ref=src, dst_ref=dst,
            send_sem=send_sems.at[i], recv_sem=recv_sems.at[i],
            device_id=peer, device_id_type=pl.DeviceIdType.MESH,
        )
        r.start()
        rdmas.append(r)
    for r in rdmas:
        r.wait()

    k_ref[...] = _f32dot(c_mine[...], wuk_ref[...]) + _f32dot(
        c_peer[...], wuk_peer[...])
    v_ref[...] = _f32dot(c_mine[...], wuv_ref[...]) + _f32dot(
        c_peer[...], wuv_peer[...])


def _proj_body(x_ref, wq_ref, wqr_ref, wkr_ref, q_ref, qr_ref, kr_ref):
    x2 = x_ref[0]
    q_ref[...] = _f32dot(x2, wq_ref[...])
    qr_ref[...] = _f32dot(x2, wqr_ref[...])
    kr_ref[...] = _f32dot(x2, wkr_ref[...])


def _attn_body(q_ref, qr_ref, kr_ref, k_ref, v_ref, o_ref):
    kr = kr_ref[...]
    for h in range(H):
        q = q_ref[:, h * Dh:(h + 1) * Dh]
        k = k_ref[:, h * Dh:(h + 1) * Dh]
        qr = qr_ref[:, h * Dr:(h + 1) * Dr]
        dn = (((1,), (1,)), ((), ()))
        s = (lax.dot_general(q, k, dn, preferred_element_type=jnp.float32)
             + lax.dot_general(qr, kr, dn, preferred_element_type=jnp.float32)
             ) * SCALE
        m = jnp.max(s, axis=1, keepdims=True)
        p = jnp.exp(s - m)
        p = p / jnp.sum(p, axis=1, keepdims=True)
        o_ref[:, h * Dh:(h + 1) * Dh] = _f32dot(p, v_ref[:, h * Dh:(h + 1) * Dh])


def _out_body(o_ref, wo_ref, out_ref):
    out_ref[0] = _f32dot(o_ref[...], wo_ref[...])


def kernel(x, Wdkv, Wuk, Wuv, Wq, Wqr, Wkr, Wo):
    f32 = jnp.float32

    K, V = pl.pallas_call(
        _kv_body,
        out_shape=[jax.ShapeDtypeStruct((S, D), f32),
                   jax.ShapeDtypeStruct((S, D), f32)],
        in_specs=[_VMEM] * 4,
        out_specs=[_VMEM, _VMEM],
        scratch_shapes=[
            pltpu.VMEM((S, DC_SHARD), f32),
            pltpu.VMEM((S, DC_SHARD), f32),
            pltpu.VMEM((DC_SHARD, D), f32),
            pltpu.VMEM((DC_SHARD, D), f32),
            pltpu.SemaphoreType.DMA((3,)),
            pltpu.SemaphoreType.DMA((3,)),
        ],
        compiler_params=pltpu.CompilerParams(
            collective_id=0, has_side_effects=True),
    )(x, Wdkv, Wuk, Wuv)

    Q, Qr, Kr = pl.pallas_call(
        _proj_body,
        out_shape=[jax.ShapeDtypeStruct((S, D), f32),
                   jax.ShapeDtypeStruct((S, H * Dr), f32),
                   jax.ShapeDtypeStruct((S, Dr), f32)],
        in_specs=[_VMEM] * 4,
        out_specs=[_VMEM] * 3,
    )(x, Wq, Wqr, Wkr)

    O = pl.pallas_call(
        _attn_body,
        out_shape=jax.ShapeDtypeStruct((S, D), f32),
        in_specs=[_VMEM] * 5,
        out_specs=_VMEM,
    )(Q, Qr, Kr, K, V)

    out = pl.pallas_call(
        _out_body,
        out_shape=jax.ShapeDtypeStruct((B, S, D), f32),
        in_specs=[_VMEM] * 2,
        out_specs=_VMEM,
    )(O, Wo)
    return out


# device time: 97465 ns/iter; 1.4110x vs baseline; 1.4110x over previous
import jax
import jax.numpy as jnp
from jax import lax
from jax.experimental import pallas as pl
from jax.experimental.pallas import tpu as pltpu

B, S, D = 1, 1024, 2048
H, Dh, Dr = 16, 128, 32
DC_SHARD = 128
SCALE = (Dh + Dr) ** -0.5

_VMEM = pl.BlockSpec(memory_space=pltpu.VMEM)
_BF = jnp.bfloat16


def _dot(a, b):
    return jnp.dot(a, b, preferred_element_type=jnp.float32)


def _dot_t(a, b):
    return lax.dot_general(a, b, (((1,), (1,)), ((), ())),
                           preferred_element_type=jnp.float32)


def _comm_proj_body(xb_ref, wdkv_ref, wuk_ref, wuv_ref, wq_ref, wqr_ref,
                    wkr_ref, q_ref, qr_ref, kr_ref, k_ref, v_ref,
                    c_mine, c_peer, wuk_peer, wuv_peer, send_sems, recv_sems):
    my_x = lax.axis_index("x")
    my_y = lax.axis_index("y")
    my_z = lax.axis_index("z")
    peer = (1 - my_x, my_y, my_z)

    barrier_sem = pltpu.get_barrier_semaphore()
    pl.semaphore_signal(barrier_sem, inc=1, device_id=peer,
                        device_id_type=pl.DeviceIdType.MESH)
    pl.semaphore_wait(barrier_sem, 1)

    xb = xb_ref[0]
    c_mine[...] = _dot(xb, wdkv_ref[...]).astype(_BF)

    rdmas = []
    for i, (src, dst) in enumerate(
        [(c_mine, c_peer), (wuk_ref, wuk_peer), (wuv_ref, wuv_peer)]
    ):
        r = pltpu.make_async_remote_copy(
            src_ref=src, dst_ref=dst,
            send_sem=send_sems.at[i], recv_sem=recv_sems.at[i],
            device_id=peer, device_id_type=pl.DeviceIdType.MESH,
        )
        r.start()
        rdmas.append(r)

    q_ref[...] = _dot(xb, wq_ref[...].astype(_BF)).astype(_BF)
    qr_ref[...] = _dot(xb, wqr_ref[...]).astype(_BF)
    kr_ref[...] = _dot(xb, wkr_ref[...]).astype(_BF)

    for r in rdmas:
        r.wait()

    k_ref[...] = (_dot(c_mine[...], wuk_ref[...])
                  + _dot(c_peer[...], wuk_peer[...])).astype(_BF)
    v_ref[...] = (_dot(c_mine[...], wuv_ref[...])
                  + _dot(c_peer[...], wuv_peer[...])).astype(_BF)


def _attn_body(q_ref, qr_ref, kr_ref, k_ref, v_ref, o_ref):
    kr = kr_ref[...]
    for h in range(H):
        q = q_ref[:, h * Dh:(h + 1) * Dh]
        k = k_ref[:, h * Dh:(h + 1) * Dh]
        qr = qr_ref[:, h * Dr:(h + 1) * Dr]
        s = (_dot_t(q, k) + _dot_t(qr, kr)) * SCALE
        m = jnp.max(s, axis=1, keepdims=True)
        p = jnp.exp(s - m)
        p = (p / jnp.sum(p, axis=1, keepdims=True)).astype(_BF)
        o_ref[:, h * Dh:(h + 1) * Dh] = _dot(
            p, v_ref[:, h * Dh:(h + 1) * Dh]).astype(_BF)


def _out_body(o_ref, wo_ref, out_ref):
    out_ref[0] = _dot(o_ref[...], wo_ref[...].astype(_BF))


def kernel(x, Wdkv, Wuk, Wuv, Wq, Wqr, Wkr, Wo):
    f32 = jnp.float32

    xb = x.astype(_BF)
    wdkv_b = Wdkv.astype(_BF)
    wuk_b = Wuk.astype(_BF)
    wuv_b = Wuv.astype(_BF)
    wqr_b = Wqr.astype(_BF)
    wkr_b = Wkr.astype(_BF)

    Q, Qr, Kr, K, V = pl.pallas_call(
        _comm_proj_body,
        out_shape=[jax.ShapeDtypeStruct((S, D), _BF),
                   jax.ShapeDtypeStruct((S, H * Dr), _BF),
                   jax.ShapeDtypeStruct((S, Dr), _BF),
                   jax.ShapeDtypeStruct((S, D), _BF),
                   jax.ShapeDtypeStruct((S, D), _BF)],
        in_specs=[_VMEM] * 7,
        out_specs=[_VMEM] * 5,
        scratch_shapes=[
            pltpu.VMEM((S, DC_SHARD), _BF),
            pltpu.VMEM((S, DC_SHARD), _BF),
            pltpu.VMEM((DC_SHARD, D), _BF),
            pltpu.VMEM((DC_SHARD, D), _BF),
            pltpu.SemaphoreType.DMA((3,)),
            pltpu.SemaphoreType.DMA((3,)),
        ],
        compiler_params=pltpu.CompilerParams(
            collective_id=0, has_side_effects=True),
    )(xb, wdkv_b, wuk_b, wuv_b, Wq, wqr_b, wkr_b)

    O = pl.pallas_call(
        _attn_body,
        out_shape=jax.ShapeDtypeStruct((S, D), _BF),
        in_specs=[_VMEM] * 5,
        out_specs=_VMEM,
    )(Q, Qr, Kr, K, V)

    out = pl.pallas_call(
        _out_body,
        out_shape=jax.ShapeDtypeStruct((B, S, D), f32),
        in_specs=[_VMEM] * 2,
        out_specs=_VMEM,
    )(O, Wo)
    return out
